# baseline (speedup 1.0000x reference)
import functools

import jax
import jax.numpy as jnp
from jax import lax
from jax.experimental import pallas as pl
from jax.experimental.pallas import tpu as pltpu
from jax.experimental.pallas import tpu_sc as plsc

_N = 8192
_S = 4096
_BN = 256
_NW = 32
_RW = _N // _NW
_CR = 8
_LANES = 16


def _rank_body(xq_ref, sx_ref, dn_ref):
    xq = xq_ref[...]
    sx = sx_ref[...]
    xs = lax.dot_general(
        xq.astype(jnp.bfloat16), sx.astype(jnp.bfloat16),
        (((1,), (0,)), ((), ())),
        preferred_element_type=jnp.float32)
    xq2 = xq[:, 0:1] ** 2 + xq[:, 1:2] ** 2 + xq[:, 2:3] ** 2
    sx2 = sx[0:1, :] ** 2 + sx[1:2, :] ** 2 + sx[2:3, :] ** 2
    dn_ref[...] = (-2.0 * xs + xq2) + sx2


_GDN = lax.GatherDimensionNumbers(
    offset_dims=(), collapsed_slice_dims=(0,), start_index_map=(0,))


def _permute(v, p):
    return lax.gather(v, p[:, None], _GDN, (1,),
                      mode=lax.GatherScatterMode.PROMISE_IN_BOUNDS)


def _sc_body(dn_hbm, x0_hbm, sx_hbm, out_hbm, dnb, sxb, qb, ddsb):
    wid = lax.axis_index("s") * 2 + lax.axis_index("c")
    base = wid * _RW
    lane = lax.broadcasted_iota(jnp.int32, (_LANES,), 0)
    inf_v = jnp.full((_LANES,), jnp.inf, jnp.float32)
    zero_i = jnp.zeros((_LANES,), jnp.int32)
    perms = [(lane ^ sh) for sh in (8, 4, 2, 1)]

    def lex_min(v, i):
        for p in perms:
            vs = _permute(v, p)
            is_ = _permute(i, p)
            take = (vs < v) | ((vs == v) & (is_ < i))
            v = jnp.where(take, vs, v)
            i = jnp.where(take, is_, i)
        return v, i

    pltpu.sync_copy(sx_hbm, sxb)
    for c in range(3):
        pltpu.sync_copy(x0_hbm.at[pl.ds(c * _N + base, _RW)],
                        qb.at[pl.ds(c * _RW, _RW)])

    def scan_row(off):
        def step(j, carry):
            m1, m2, m3, i1, i2, i3 = carry
            v = dnb[pl.ds(off + j * _LANES, _LANES)]
            jv = lane + j * _LANES
            b1 = v < m1
            b2 = v < m2
            b3 = v < m3
            m3n = jnp.where(b3, jnp.where(b2, m2, v), m3)
            i3n = jnp.where(b3, jnp.where(b2, i2, jv), i3)
            m2n = jnp.where(b2, jnp.where(b1, m1, v), m2)
            i2n = jnp.where(b2, jnp.where(b1, i1, jv), i2)
            m1n = jnp.where(b1, v, m1)
            i1n = jnp.where(b1, jv, i1)
            return m1n, m2n, m3n, i1n, i2n, i3n

        return lax.fori_loop(
            0, _S // _LANES, step,
            (inf_v, inf_v, inf_v, zero_i, zero_i, zero_i))

    def do_chunk(ch, _):
        pltpu.sync_copy(
            dn_hbm.at[pl.ds((base + ch * _CR) * _S, _CR * _S)], dnb)

        def do_row(r8, _):
            m1, m2, m3, i1, i2, i3 = scan_row(r8 * _S)
            ri = ch * _CR + r8
            gis = []
            for k in range(3):
                g, gi = lex_min(m1, i1)
                gis.append(gi)
                pop = (m1 == g) & (i1 == gi)
                m1 = jnp.where(pop, m2, m1)
                i1 = jnp.where(pop, i2, i1)
                m2 = jnp.where(pop, m3, m2)
                i2 = jnp.where(pop, i3, i2)
                m3 = jnp.where(pop, inf_v, m3)
            iv = jnp.where(lane == 0, gis[0],
                           jnp.where(lane == 1, gis[1],
                                     jnp.where(lane == 2, gis[2], 0)))
            gx = plsc.load_gather(sxb, [iv])
            gy = plsc.load_gather(sxb, [iv + _S])
            gz = plsc.load_gather(sxb, [iv + 2 * _S])
            qx = plsc.load_gather(qb, [zero_i + ri])
            qy = plsc.load_gather(qb, [zero_i + (_RW + ri)])
            qz = plsc.load_gather(qb, [zero_i + (2 * _RW + ri)])
            ex = qx - gx
            ey = qy - gy
            ez = qz - gz
            dd = ex * ex + ey * ey + ez * ez
            plsc.store_scatter(ddsb, [ri * 3 + jnp.minimum(lane, 2)], dd,
                               mask=lane < 3)
            return 0

        lax.fori_loop(0, _CR, do_row, 0)
        return 0

    lax.fori_loop(0, _RW // _CR, do_chunk, 0)
    pltpu.sync_copy(ddsb, out_hbm.at[wid])


def _out_body(dd_ref, ft_ref, out_ref):
    inv1 = 1.0 / jnp.maximum(jnp.sqrt(dd_ref[:, 0:1]), 1e-10)
    inv2 = 1.0 / jnp.maximum(jnp.sqrt(dd_ref[:, 1:2]), 1e-10)
    inv3 = 1.0 / jnp.maximum(jnp.sqrt(dd_ref[:, 2:3]), 1e-10)
    norm = inv1 + inv2 + inv3
    w = jnp.concatenate([inv1, inv2, inv3], axis=1) / norm
    out_ref[...] = lax.dot_general(
        ft_ref[...], w, (((1,), (1,)), ((), ())),
        preferred_element_type=jnp.float32)


def kernel(xyz, sparse_xyz, sparse_frame):
    xq = jnp.transpose(xyz[0])
    x0 = xyz[0]
    sx = sparse_xyz[0]
    ft = jnp.transpose(sparse_frame[0])

    dn = pl.pallas_call(
        _rank_body,
        grid=(_N // _BN,),
        in_specs=[
            pl.BlockSpec((_BN, 3), lambda i: (i, 0)),
            pl.BlockSpec((3, _S), lambda i: (0, 0)),
        ],
        out_specs=pl.BlockSpec((_BN, _S), lambda i: (i, 0)),
        out_shape=jax.ShapeDtypeStruct((_N, _S), jnp.float32),
    )(xq, sx)

    sc_fn = pl.kernel(
        _sc_body,
        mesh=plsc.VectorSubcoreMesh(core_axis_name="c", subcore_axis_name="s"),
        compiler_params=pltpu.CompilerParams(needs_layout_passes=False),
        out_type=jax.ShapeDtypeStruct((_NW, 3 * _RW), jnp.float32),
        scratch_types=[
            pltpu.VMEM((_CR * _S,), jnp.float32),
            pltpu.VMEM((3 * _S,), jnp.float32),
            pltpu.VMEM((3 * _RW,), jnp.float32),
            pltpu.VMEM((3 * _RW,), jnp.float32),
        ],
    )
    dds = sc_fn(dn.reshape(_N * _S), x0.reshape(3 * _N), sx.reshape(3 * _S))
    dd3 = dds.reshape(_N, 3)

    out = pl.pallas_call(
        _out_body,
        grid=(_N // _BN,),
        in_specs=[
            pl.BlockSpec((_BN, 3), lambda i: (i, 0)),
            pl.BlockSpec((_S, 3), lambda i: (0, 0)),
        ],
        out_specs=pl.BlockSpec((_S, _BN), lambda i: (0, i)),
        out_shape=jax.ShapeDtypeStruct((_S, _N), jnp.float32),
    )(dd3, ft)
    return out[None]


# speedup vs baseline: 1.3702x; 1.3702x over previous
import functools

import jax
import jax.numpy as jnp
from jax import lax
from jax.experimental import pallas as pl
from jax.experimental.pallas import tpu as pltpu
from jax.experimental.pallas import tpu_sc as plsc

_N = 8192
_S = 4096
_BN = 256
_NW = 32
_RW = _N // _NW
_CR = 8
_LANES = 16


def _rank_body(xq_ref, sx_ref, dn_ref):
    xq = xq_ref[...]
    sx = sx_ref[...]
    xs = lax.dot_general(
        xq.astype(jnp.bfloat16), sx.astype(jnp.bfloat16),
        (((1,), (0,)), ((), ())),
        preferred_element_type=jnp.float32)
    xq2 = xq[:, 0:1] ** 2 + xq[:, 1:2] ** 2 + xq[:, 2:3] ** 2
    sx2 = sx[0:1, :] ** 2 + sx[1:2, :] ** 2 + sx[2:3, :] ** 2
    dn_ref[...] = (-2.0 * xs + xq2) + sx2


_GDN = lax.GatherDimensionNumbers(
    offset_dims=(), collapsed_slice_dims=(0,), start_index_map=(0,))


def _permute(v, p):
    return lax.gather(v, p[:, None], _GDN, (1,),
                      mode=lax.GatherScatterMode.PROMISE_IN_BOUNDS)


def _sc_body(dn_hbm, x0_hbm, sx_hbm, out_hbm, dnb, sxb, qb, ddsb):
    wid = lax.axis_index("s") * 2 + lax.axis_index("c")
    base = wid * _RW
    lane = lax.broadcasted_iota(jnp.int32, (_LANES,), 0)
    inf_v = jnp.full((_LANES,), jnp.inf, jnp.float32)
    zero_i = jnp.zeros((_LANES,), jnp.int32)
    perms = [(lane ^ sh) for sh in (8, 4, 2, 1)]

    def lex_min(v, i):
        for p in perms:
            vs = _permute(v, p)
            is_ = _permute(i, p)
            take = (vs < v) | ((vs == v) & (is_ < i))
            v = jnp.where(take, vs, v)
            i = jnp.where(take, is_, i)
        return v, i

    pltpu.sync_copy(sx_hbm, sxb)
    for c in range(3):
        pltpu.sync_copy(x0_hbm.at[pl.ds(c * _N + base, _RW)],
                        qb.at[pl.ds(c * _RW, _RW)])

    def scan_row(r8):
        def step(j, carry):
            m1, m2, m3, i1, i2, i3 = carry
            v = dnb[r8, pl.ds(j * _LANES, _LANES)]
            jv = lane + j * _LANES
            b1 = v < m1
            b2 = v < m2
            b3 = v < m3
            m3n = jnp.where(b3, jnp.where(b2, m2, v), m3)
            i3n = jnp.where(b3, jnp.where(b2, i2, jv), i3)
            m2n = jnp.where(b2, jnp.where(b1, m1, v), m2)
            i2n = jnp.where(b2, jnp.where(b1, i1, jv), i2)
            m1n = jnp.where(b1, v, m1)
            i1n = jnp.where(b1, jv, i1)
            return m1n, m2n, m3n, i1n, i2n, i3n

        return lax.fori_loop(
            0, _S // _LANES, step,
            (inf_v, inf_v, inf_v, zero_i, zero_i, zero_i),
            unroll=8)

    def do_chunk(ch, _):
        pltpu.sync_copy(
            dn_hbm.at[pl.ds(base + ch * _CR, _CR), :], dnb)

        def do_row(r8, _):
            m1, m2, m3, i1, i2, i3 = scan_row(r8)
            ri = ch * _CR + r8
            gis = []
            for k in range(3):
                g, gi = lex_min(m1, i1)
                gis.append(gi)
                pop = (m1 == g) & (i1 == gi)
                m1 = jnp.where(pop, m2, m1)
                i1 = jnp.where(pop, i2, i1)
                m2 = jnp.where(pop, m3, m2)
                i2 = jnp.where(pop, i3, i2)
                m3 = jnp.where(pop, inf_v, m3)
            iv = jnp.where(lane == 0, gis[0],
                           jnp.where(lane == 1, gis[1],
                                     jnp.where(lane == 2, gis[2], 0)))
            gx = plsc.load_gather(sxb, [iv])
            gy = plsc.load_gather(sxb, [iv + _S])
            gz = plsc.load_gather(sxb, [iv + 2 * _S])
            qx = plsc.load_gather(qb, [zero_i + ri])
            qy = plsc.load_gather(qb, [zero_i + (_RW + ri)])
            qz = plsc.load_gather(qb, [zero_i + (2 * _RW + ri)])
            ex = qx - gx
            ey = qy - gy
            ez = qz - gz
            dd = ex * ex + ey * ey + ez * ez
            plsc.store_scatter(ddsb, [ri * 3 + jnp.minimum(lane, 2)], dd,
                               mask=lane < 3)
            return 0

        lax.fori_loop(0, _CR, do_row, 0)
        return 0

    lax.fori_loop(0, _RW // _CR, do_chunk, 0)
    pltpu.sync_copy(ddsb, out_hbm.at[wid])


def _out_body(dd_ref, ft_ref, out_ref):
    inv1 = 1.0 / jnp.maximum(jnp.sqrt(dd_ref[:, 0:1]), 1e-10)
    inv2 = 1.0 / jnp.maximum(jnp.sqrt(dd_ref[:, 1:2]), 1e-10)
    inv3 = 1.0 / jnp.maximum(jnp.sqrt(dd_ref[:, 2:3]), 1e-10)
    norm = inv1 + inv2 + inv3
    w = jnp.concatenate([inv1, inv2, inv3], axis=1) / norm
    out_ref[...] = lax.dot_general(
        ft_ref[...], w, (((1,), (1,)), ((), ())),
        preferred_element_type=jnp.float32)


def kernel(xyz, sparse_xyz, sparse_frame):
    xq = jnp.transpose(xyz[0])
    x0 = xyz[0]
    sx = sparse_xyz[0]
    ft = jnp.transpose(sparse_frame[0])

    dn = pl.pallas_call(
        _rank_body,
        grid=(_N // _BN,),
        in_specs=[
            pl.BlockSpec((_BN, 3), lambda i: (i, 0)),
            pl.BlockSpec((3, _S), lambda i: (0, 0)),
        ],
        out_specs=pl.BlockSpec((_BN, _S), lambda i: (i, 0)),
        out_shape=jax.ShapeDtypeStruct((_N, _S), jnp.float32),
    )(xq, sx)

    sc_fn = pl.kernel(
        _sc_body,
        mesh=plsc.VectorSubcoreMesh(core_axis_name="c", subcore_axis_name="s"),
        compiler_params=pltpu.CompilerParams(needs_layout_passes=False),
        out_type=jax.ShapeDtypeStruct((_NW, 3 * _RW), jnp.float32),
        scratch_types=[
            pltpu.VMEM((_CR, _S), jnp.float32),
            pltpu.VMEM((3 * _S,), jnp.float32),
            pltpu.VMEM((3 * _RW,), jnp.float32),
            pltpu.VMEM((3 * _RW,), jnp.float32),
        ],
    )
    dds = sc_fn(dn, x0.reshape(3 * _N), sx.reshape(3 * _S))
    dd3 = dds.reshape(_N, 3)

    out = pl.pallas_call(
        _out_body,
        grid=(_N // _BN,),
        in_specs=[
            pl.BlockSpec((_BN, 3), lambda i: (i, 0)),
            pl.BlockSpec((_S, 3), lambda i: (0, 0)),
        ],
        out_specs=pl.BlockSpec((_S, _BN), lambda i: (0, i)),
        out_shape=jax.ShapeDtypeStruct((_S, _N), jnp.float32),
    )(dd3, ft)
    return out[None]


# speedup vs baseline: 1.4179x; 1.0347x over previous
import functools

import jax
import jax.numpy as jnp
from jax import lax
from jax.experimental import pallas as pl
from jax.experimental.pallas import tpu as pltpu
from jax.experimental.pallas import tpu_sc as plsc

_N = 8192
_S = 4096
_BN = 256
_NW = 32
_RW = _N // _NW
_CR = 8
_LANES = 16


def _rank_body(xq_ref, sx_ref, dn_ref):
    xq = xq_ref[...]
    sx = sx_ref[...]
    xs = lax.dot_general(
        xq.astype(jnp.bfloat16), sx.astype(jnp.bfloat16),
        (((1,), (0,)), ((), ())),
        preferred_element_type=jnp.float32)
    xq2 = xq[:, 0:1] ** 2 + xq[:, 1:2] ** 2 + xq[:, 2:3] ** 2
    sx2 = sx[0:1, :] ** 2 + sx[1:2, :] ** 2 + sx[2:3, :] ** 2
    dn_ref[...] = (-2.0 * xs + xq2) + sx2


_GDN = lax.GatherDimensionNumbers(
    offset_dims=(), collapsed_slice_dims=(0,), start_index_map=(0,))


def _permute(v, p):
    return lax.gather(v, p[:, None], _GDN, (1,),
                      mode=lax.GatherScatterMode.PROMISE_IN_BOUNDS)


def _sc_body(dn_hbm, x0_hbm, sx_hbm, out_hbm, dnb, sxb, qb, ddsb):
    wid = lax.axis_index("s") * 2 + lax.axis_index("c")
    base = wid * _RW
    lane = lax.broadcasted_iota(jnp.int32, (_LANES,), 0)
    inf_v = jnp.full((_LANES,), jnp.inf, jnp.float32)
    zero_i = jnp.zeros((_LANES,), jnp.int32)
    perms = [(lane ^ sh) for sh in (8, 4, 2, 1)]

    def lex_min(v, i):
        for p in perms:
            vs = _permute(v, p)
            is_ = _permute(i, p)
            take = (vs < v) | ((vs == v) & (is_ < i))
            v = jnp.where(take, vs, v)
            i = jnp.where(take, is_, i)
        return v, i

    pltpu.sync_copy(sx_hbm, sxb)
    for c in range(3):
        pltpu.sync_copy(x0_hbm.at[pl.ds(c * _N + base, _RW)],
                        qb.at[pl.ds(c * _RW, _RW)])

    def _insert(carry, v, jv):
        m1, m2, m3, i1, i2, i3 = carry
        b1 = v < m1
        b2 = v < m2
        b3 = v < m3
        m3n = jnp.where(b3, jnp.where(b2, m2, v), m3)
        i3n = jnp.where(b3, jnp.where(b2, i2, jv), i3)
        m2n = jnp.where(b2, jnp.where(b1, m1, v), m2)
        i2n = jnp.where(b2, jnp.where(b1, i1, jv), i2)
        m1n = jnp.where(b1, v, m1)
        i1n = jnp.where(b1, jv, i1)
        return m1n, m2n, m3n, i1n, i2n, i3n

    def scan_row_pair(ra, rb):
        def step(j, carry):
            ca, cb = carry
            va = dnb[ra, pl.ds(j * _LANES, _LANES)]
            vb = dnb[rb, pl.ds(j * _LANES, _LANES)]
            jv = lane + j * _LANES
            return _insert(ca, va, jv), _insert(cb, vb, jv)

        init = (inf_v, inf_v, inf_v, zero_i, zero_i, zero_i)
        return lax.fori_loop(0, _S // _LANES, step, (init, init), unroll=4)

    def do_chunk(ch, _):
        pltpu.sync_copy(
            dn_hbm.at[pl.ds(base + ch * _CR, _CR), :], dnb)

        def finish(ri, carry):
            m1, m2, m3, i1, i2, i3 = carry
            gis = []
            for k in range(3):
                g, gi = lex_min(m1, i1)
                gis.append(gi)
                pop = (m1 == g) & (i1 == gi)
                m1 = jnp.where(pop, m2, m1)
                i1 = jnp.where(pop, i2, i1)
                m2 = jnp.where(pop, m3, m2)
                i2 = jnp.where(pop, i3, i2)
                m3 = jnp.where(pop, inf_v, m3)
            iv = jnp.where(lane == 0, gis[0],
                           jnp.where(lane == 1, gis[1],
                                     jnp.where(lane == 2, gis[2], 0)))
            gx = plsc.load_gather(sxb, [iv])
            gy = plsc.load_gather(sxb, [iv + _S])
            gz = plsc.load_gather(sxb, [iv + 2 * _S])
            qx = plsc.load_gather(qb, [zero_i + ri])
            qy = plsc.load_gather(qb, [zero_i + (_RW + ri)])
            qz = plsc.load_gather(qb, [zero_i + (2 * _RW + ri)])
            ex = qx - gx
            ey = qy - gy
            ez = qz - gz
            dd = ex * ex + ey * ey + ez * ez
            plsc.store_scatter(ddsb, [ri * 3 + jnp.minimum(lane, 2)], dd,
                               mask=lane < 3)

        def do_pair(r4, _):
            ca, cb = scan_row_pair(2 * r4, 2 * r4 + 1)
            finish(ch * _CR + 2 * r4, ca)
            finish(ch * _CR + 2 * r4 + 1, cb)
            return 0

        lax.fori_loop(0, _CR // 2, do_pair, 0)
        return 0

    lax.fori_loop(0, _RW // _CR, do_chunk, 0)
    pltpu.sync_copy(ddsb, out_hbm.at[wid])


def _out_body(dd_ref, ft_ref, out_ref):
    inv1 = 1.0 / jnp.maximum(jnp.sqrt(dd_ref[:, 0:1]), 1e-10)
    inv2 = 1.0 / jnp.maximum(jnp.sqrt(dd_ref[:, 1:2]), 1e-10)
    inv3 = 1.0 / jnp.maximum(jnp.sqrt(dd_ref[:, 2:3]), 1e-10)
    norm = inv1 + inv2 + inv3
    w = jnp.concatenate([inv1, inv2, inv3], axis=1) / norm
    out_ref[...] = lax.dot_general(
        ft_ref[...], w, (((1,), (1,)), ((), ())),
        preferred_element_type=jnp.float32)


def kernel(xyz, sparse_xyz, sparse_frame):
    xq = jnp.transpose(xyz[0])
    x0 = xyz[0]
    sx = sparse_xyz[0]
    ft = jnp.transpose(sparse_frame[0])

    dn = pl.pallas_call(
        _rank_body,
        grid=(_N // _BN,),
        in_specs=[
            pl.BlockSpec((_BN, 3), lambda i: (i, 0)),
            pl.BlockSpec((3, _S), lambda i: (0, 0)),
        ],
        out_specs=pl.BlockSpec((_BN, _S), lambda i: (i, 0)),
        out_shape=jax.ShapeDtypeStruct((_N, _S), jnp.float32),
    )(xq, sx)

    sc_fn = pl.kernel(
        _sc_body,
        mesh=plsc.VectorSubcoreMesh(core_axis_name="c", subcore_axis_name="s"),
        compiler_params=pltpu.CompilerParams(needs_layout_passes=False),
        out_type=jax.ShapeDtypeStruct((_NW, 3 * _RW), jnp.float32),
        scratch_types=[
            pltpu.VMEM((_CR, _S), jnp.float32),
            pltpu.VMEM((3 * _S,), jnp.float32),
            pltpu.VMEM((3 * _RW,), jnp.float32),
            pltpu.VMEM((3 * _RW,), jnp.float32),
        ],
    )
    dds = sc_fn(dn, x0.reshape(3 * _N), sx.reshape(3 * _S))
    dd3 = dds.reshape(_N, 3)

    out = pl.pallas_call(
        _out_body,
        grid=(_N // _BN,),
        in_specs=[
            pl.BlockSpec((_BN, 3), lambda i: (i, 0)),
            pl.BlockSpec((_S, 3), lambda i: (0, 0)),
        ],
        out_specs=pl.BlockSpec((_S, _BN), lambda i: (0, i)),
        out_shape=jax.ShapeDtypeStruct((_S, _N), jnp.float32),
    )(dd3, ft)
    return out[None]


# speedup vs baseline: 1.6814x; 1.1859x over previous
import functools

import jax
import jax.numpy as jnp
from jax import lax
from jax.experimental import pallas as pl
from jax.experimental.pallas import tpu as pltpu
from jax.experimental.pallas import tpu_sc as plsc

_N = 8192
_S = 4096
_BN = 256
_NW = 32
_RW = _N // _NW
_CR = 8
_LANES = 16


def _rank_body(xq_ref, sx_ref, dn_ref):
    xq = xq_ref[...]
    sx = sx_ref[...]
    xs = lax.dot_general(
        xq.astype(jnp.bfloat16), sx.astype(jnp.bfloat16),
        (((1,), (0,)), ((), ())),
        preferred_element_type=jnp.float32)
    xq2 = xq[:, 0:1] ** 2 + xq[:, 1:2] ** 2 + xq[:, 2:3] ** 2
    sx2 = sx[0:1, :] ** 2 + sx[1:2, :] ** 2 + sx[2:3, :] ** 2
    dn_ref[...] = (-2.0 * xs + xq2) + sx2


_GDN = lax.GatherDimensionNumbers(
    offset_dims=(), collapsed_slice_dims=(0,), start_index_map=(0,))


def _permute(v, p):
    return lax.gather(v, p[:, None], _GDN, (1,),
                      mode=lax.GatherScatterMode.PROMISE_IN_BOUNDS)


def _sc_body(dn_hbm, x0_hbm, sx_hbm, out_hbm, dnb0, dnb1, sxb, qb, ddsb,
             sem0, sem1):
    wid = lax.axis_index("s") * 2 + lax.axis_index("c")
    base = wid * _RW
    lane = lax.broadcasted_iota(jnp.int32, (_LANES,), 0)
    inf_v = jnp.full((_LANES,), jnp.inf, jnp.float32)
    zero_i = jnp.zeros((_LANES,), jnp.int32)
    perms = [(lane ^ sh) for sh in (8, 4, 2, 1)]

    def lex_min(v, i):
        for p in perms:
            vs = _permute(v, p)
            is_ = _permute(i, p)
            take = (vs < v) | ((vs == v) & (is_ < i))
            v = jnp.where(take, vs, v)
            i = jnp.where(take, is_, i)
        return v, i

    pltpu.sync_copy(sx_hbm, sxb)
    for c in range(3):
        pltpu.sync_copy(x0_hbm.at[pl.ds(c * _N + base, _RW)],
                        qb.at[pl.ds(c * _RW, _RW)])

    def _insert(carry, v, jv):
        m1, m2, m3, i1, i2, i3 = carry
        b1 = v < m1
        b2 = v < m2
        b3 = v < m3
        m3n = jnp.where(b3, jnp.where(b2, m2, v), m3)
        i3n = jnp.where(b3, jnp.where(b2, i2, jv), i3)
        m2n = jnp.where(b2, jnp.where(b1, m1, v), m2)
        i2n = jnp.where(b2, jnp.where(b1, i1, jv), i2)
        m1n = jnp.where(b1, v, m1)
        i1n = jnp.where(b1, jv, i1)
        return m1n, m2n, m3n, i1n, i2n, i3n

    def scan_row_pair(dnb, ra, rb):
        def step(j, carry):
            ca, cb = carry
            va = dnb[ra, pl.ds(j * _LANES, _LANES)]
            vb = dnb[rb, pl.ds(j * _LANES, _LANES)]
            jv = lane + j * _LANES
            return _insert(ca, va, jv), _insert(cb, vb, jv)

        init = (inf_v, inf_v, inf_v, zero_i, zero_i, zero_i)
        return lax.fori_loop(0, _S // _LANES, step, (init, init), unroll=4)

    if True:
        def finish(ri, carry):
            m1, m2, m3, i1, i2, i3 = carry
            gis = []
            for k in range(3):
                g, gi = lex_min(m1, i1)
                gis.append(gi)
                pop = (m1 == g) & (i1 == gi)
                m1 = jnp.where(pop, m2, m1)
                i1 = jnp.where(pop, i2, i1)
                m2 = jnp.where(pop, m3, m2)
                i2 = jnp.where(pop, i3, i2)
                m3 = jnp.where(pop, inf_v, m3)
            iv = jnp.where(lane == 0, gis[0],
                           jnp.where(lane == 1, gis[1],
                                     jnp.where(lane == 2, gis[2], 0)))
            gx = plsc.load_gather(sxb, [iv])
            gy = plsc.load_gather(sxb, [iv + _S])
            gz = plsc.load_gather(sxb, [iv + 2 * _S])
            qx = plsc.load_gather(qb, [zero_i + ri])
            qy = plsc.load_gather(qb, [zero_i + (_RW + ri)])
            qz = plsc.load_gather(qb, [zero_i + (2 * _RW + ri)])
            ex = qx - gx
            ey = qy - gy
            ez = qz - gz
            dd = ex * ex + ey * ey + ez * ez
            plsc.store_scatter(ddsb, [ri * 3 + jnp.minimum(lane, 2)], dd,
                               mask=lane < 3)

    def process(dnb, ch):
        def do_pair(r4, _):
            ca, cb = scan_row_pair(dnb, 2 * r4, 2 * r4 + 1)
            finish(ch * _CR + 2 * r4, ca)
            finish(ch * _CR + 2 * r4 + 1, cb)
            return 0

        lax.fori_loop(0, _CR // 2, do_pair, 0)

    nch = _RW // _CR
    pltpu.async_copy(dn_hbm.at[pl.ds(base, _CR), :], dnb0, sem0)

    def outer(p, _):
        even = 2 * p
        pltpu.make_async_copy(dn_hbm.at[pl.ds(0, _CR), :], dnb0, sem0).wait()
        pltpu.async_copy(
            dn_hbm.at[pl.ds(base + (even + 1) * _CR, _CR), :], dnb1, sem1)
        process(dnb0, even)
        pltpu.make_async_copy(dn_hbm.at[pl.ds(0, _CR), :], dnb1, sem1).wait()

        @pl.when(p < nch // 2 - 1)
        def _():
            pltpu.async_copy(
                dn_hbm.at[pl.ds(base + (even + 2) * _CR, _CR), :], dnb0, sem0)

        process(dnb1, even + 1)
        return 0

    lax.fori_loop(0, nch // 2, outer, 0)
    pltpu.sync_copy(ddsb, out_hbm.at[wid])


def _out_body(dd_ref, ft_ref, out_ref):
    inv1 = 1.0 / jnp.maximum(jnp.sqrt(dd_ref[:, 0:1]), 1e-10)
    inv2 = 1.0 / jnp.maximum(jnp.sqrt(dd_ref[:, 1:2]), 1e-10)
    inv3 = 1.0 / jnp.maximum(jnp.sqrt(dd_ref[:, 2:3]), 1e-10)
    norm = inv1 + inv2 + inv3
    w = jnp.concatenate([inv1, inv2, inv3], axis=1) / norm
    out_ref[...] = lax.dot_general(
        ft_ref[...], w, (((1,), (1,)), ((), ())),
        preferred_element_type=jnp.float32)


def kernel(xyz, sparse_xyz, sparse_frame):
    xq = jnp.transpose(xyz[0])
    x0 = xyz[0]
    sx = sparse_xyz[0]
    ft = jnp.transpose(sparse_frame[0])

    dn = pl.pallas_call(
        _rank_body,
        grid=(_N // _BN,),
        in_specs=[
            pl.BlockSpec((_BN, 3), lambda i: (i, 0)),
            pl.BlockSpec((3, _S), lambda i: (0, 0)),
        ],
        out_specs=pl.BlockSpec((_BN, _S), lambda i: (i, 0)),
        out_shape=jax.ShapeDtypeStruct((_N, _S), jnp.float32),
    )(xq, sx)

    sc_fn = pl.kernel(
        _sc_body,
        mesh=plsc.VectorSubcoreMesh(core_axis_name="c", subcore_axis_name="s"),
        compiler_params=pltpu.CompilerParams(needs_layout_passes=False),
        out_type=jax.ShapeDtypeStruct((_NW, 3 * _RW), jnp.float32),
        scratch_types=[
            pltpu.VMEM((_CR, _S), jnp.float32),
            pltpu.VMEM((_CR, _S), jnp.float32),
            pltpu.VMEM((3 * _S,), jnp.float32),
            pltpu.VMEM((3 * _RW,), jnp.float32),
            pltpu.VMEM((3 * _RW,), jnp.float32),
            pltpu.SemaphoreType.DMA,
            pltpu.SemaphoreType.DMA,
        ],
    )
    dds = sc_fn(dn, x0.reshape(3 * _N), sx.reshape(3 * _S))
    dd3 = dds.reshape(_N, 3)

    out = pl.pallas_call(
        _out_body,
        grid=(_N // _BN,),
        in_specs=[
            pl.BlockSpec((_BN, 3), lambda i: (i, 0)),
            pl.BlockSpec((_S, 3), lambda i: (0, 0)),
        ],
        out_specs=pl.BlockSpec((_S, _BN), lambda i: (0, i)),
        out_shape=jax.ShapeDtypeStruct((_S, _N), jnp.float32),
    )(dd3, ft)
    return out[None]
